# baseline (speedup 1.0000x reference)
import functools

import jax
import jax.numpy as jnp
from jax import lax
from jax.experimental import pallas as pl
from jax.experimental.pallas import tpu as pltpu
from jax.experimental.pallas import tpu_sc as plsc

_B = 8
_SEQ = 512
_TOT = 2048
_D = 128
_CLS = 101
_SEP = 102
_PAD = 0

_NC = 2
_NS = 16
_NW = _NC * _NS
_CHUNK = _B * _SEQ // _NW
_CPB = _SEQ // _CHUNK
_TWIN = 160
_ZBASE = 2 * _TOT

_BUDGET = _SEQ - 3
_HALF = _BUDGET // 2


def _sc_body(tok1_hbm, tok2_hbm, ctab_hbm, cu1_hbm, cu2_hbm,
             ids_hbm, mask_hbm, types_hbm, packed_hbm,
             cu1_v, cu2_v, tokw1, tokw2, cidx_v,
             buf, ids_v, mask_v, types_v, sem):
    wid = lax.axis_index("s") * _NC + lax.axis_index("c")
    b = wid // _CPB
    base = (wid % _CPB) * _CHUNK

    pltpu.sync_copy(cu1_hbm, cu1_v)
    pltpu.sync_copy(cu2_hbm, cu2_v)

    iota = lax.iota(jnp.int32, 16)
    lo = jnp.minimum(iota, _B)
    hi = jnp.minimum(iota + 1, _B)
    s1v = plsc.load_gather(cu1_v, [lo])
    s2v = plsc.load_gather(cu2_v, [lo])
    l1v = plsc.load_gather(cu1_v, [hi]) - s1v
    l2v = plsc.load_gather(cu2_v, [hi]) - s2v

    over = (l1v + l2v) > _BUDGET
    t1 = jnp.where(l2v <= _HALF, _BUDGET - l2v,
                   jnp.where(l1v <= _HALF, l1v, _HALF))
    t2 = jnp.where(l2v <= _HALF, l2v,
                   jnp.where(l1v <= _HALF, _BUDGET - l1v, _BUDGET - _HALF))
    l1v = jnp.where(over, t1, l1v)
    l2v = jnp.where(over, t2, l2v)

    sel = iota == b
    l1 = jnp.sum(jnp.where(sel, l1v, 0))
    l2 = jnp.sum(jnp.where(sel, l2v, 0))
    s1 = jnp.sum(jnp.where(sel, s1v, 0))
    s2 = jnp.sum(jnp.where(sel, s2v, 0))

    w1 = pl.multiple_of(jnp.clip((s1 + base - 1) & -16, 0, _TOT - _TWIN), 16)
    w2 = pl.multiple_of(
        jnp.clip((s2 + base - l1 - 2) & -16, 0, _TOT - _TWIN), 16)
    c1 = pltpu.async_copy(tok1_hbm.at[pl.ds(w1, _TWIN)], tokw1, sem)
    c2 = pltpu.async_copy(tok2_hbm.at[pl.ds(w2, _TWIN)], tokw2, sem)

    for j in range(_CHUNK // 16):
        p = base + j * 16 + iota
        in1 = (p >= 1) & (p <= l1)
        in2 = (p >= l1 + 2) & (p <= l1 + 1 + l2)
        cidx_v[pl.ds(j * 16, 16)] = jnp.where(
            in1, s1 + p - 1,
            jnp.where(in2, _TOT + s2 + p - l1 - 2,
                      _ZBASE + j * 16 + iota))

    g = pltpu.async_copy(ctab_hbm.at[cidx_v], buf, sem)

    c1.wait()
    c2.wait()

    for j in range(_CHUNK // 16):
        p = base + j * 16 + iota
        in1 = (p >= 1) & (p <= l1)
        in2 = (p >= l1 + 2) & (p <= l1 + 1 + l2)
        sep = (p == l1 + 1) | (p == l1 + l2 + 2)
        lt1 = jnp.clip(s1 + p - 1 - w1, 0, _TWIN - 1)
        lt2 = jnp.clip(s2 + p - l1 - 2 - w2, 0, _TWIN - 1)
        t1g = plsc.load_gather(tokw1, [lt1])
        t2g = plsc.load_gather(tokw2, [lt2])
        ids = jnp.where(p == 0, _CLS,
                        jnp.where(sep, _SEP,
                                  jnp.where(in1, t1g,
                                            jnp.where(in2, t2g, _PAD))))
        ids_v[pl.ds(j * 16, 16)] = ids
        mask_v[pl.ds(j * 16, 16)] = (p < l1 + l2 + 3).astype(jnp.int32)
        types_v[pl.ds(j * 16, 16)] = (
            (p >= l1 + 2) & (p <= l1 + l2 + 2)).astype(jnp.int32)

    pltpu.sync_copy(ids_v, ids_hbm.at[b, pl.ds(base, _CHUNK)])
    pltpu.sync_copy(mask_v, mask_hbm.at[b, pl.ds(base, _CHUNK)])
    pltpu.sync_copy(types_v, types_hbm.at[b, pl.ds(base, _CHUNK)])

    g.wait()
    pltpu.sync_copy(buf, packed_hbm.at[b, pl.ds(base, _CHUNK)])


_sc_call = functools.partial(
    pl.kernel,
    out_type=(
        jax.ShapeDtypeStruct((_B, _SEQ), jnp.int32),
        jax.ShapeDtypeStruct((_B, _SEQ), jnp.int32),
        jax.ShapeDtypeStruct((_B, _SEQ), jnp.int32),
        jax.ShapeDtypeStruct((_B, _SEQ, _D), jnp.float32),
    ),
    mesh=plsc.VectorSubcoreMesh(
        core_axis_name="c", subcore_axis_name="s",
        num_cores=_NC, num_subcores=_NS),
    compiler_params=pltpu.CompilerParams(
        needs_layout_passes=False, use_tc_tiling_on_sc=False),
    scratch_types=[
        pltpu.VMEM((_B + 1,), jnp.int32),
        pltpu.VMEM((_B + 1,), jnp.int32),
        pltpu.VMEM((_TWIN,), jnp.int32),
        pltpu.VMEM((_TWIN,), jnp.int32),
        pltpu.VMEM((_CHUNK,), jnp.int32),
        pltpu.VMEM((_CHUNK, _D), jnp.float32),
        pltpu.VMEM((_CHUNK,), jnp.int32),
        pltpu.VMEM((_CHUNK,), jnp.int32),
        pltpu.VMEM((_CHUNK,), jnp.int32),
        pltpu.SemaphoreType.DMA,
    ],
)(_sc_body)


def kernel(tokens1, tokens2, feats1, feats2, cu_seqlens1, cu_seqlens2):
    ctab = jnp.concatenate(
        [feats1, feats2, jnp.zeros((_CHUNK, _D), jnp.float32)], axis=0)
    return _sc_call(tokens1.astype(jnp.int32), tokens2.astype(jnp.int32),
                    ctab,
                    cu_seqlens1.astype(jnp.int32),
                    cu_seqlens2.astype(jnp.int32))


# speedup vs baseline: 1.1129x; 1.1129x over previous
import functools

import jax
import jax.numpy as jnp
from jax import lax
from jax.experimental import pallas as pl
from jax.experimental.pallas import tpu as pltpu
from jax.experimental.pallas import tpu_sc as plsc

_B = 8
_SEQ = 512
_TOT = 2048
_D = 128
_CLS = 101
_SEP = 102
_PAD = 0

_NC = 2
_NS = 16
_NW = _NC * _NS
_CHUNK = _B * _SEQ // _NW
_CPB = _SEQ // _CHUNK
_TWIN = 160
_ZBASE = 2 * _TOT

_BUDGET = _SEQ - 3
_HALF = _BUDGET // 2


def _sc_body(tok1_hbm, tok2_hbm, ctab_hbm, cu1_hbm, cu2_hbm,
             ids_hbm, mask_hbm, types_hbm, packed_hbm,
             cu1_v, cu2_v, tokw1, tokw2, cidx_v,
             buf, ids_v, mask_v, types_v, sem):
    wid = lax.axis_index("s") * _NC + lax.axis_index("c")
    b = wid // _CPB
    base = (wid % _CPB) * _CHUNK

    pltpu.sync_copy(cu1_hbm, cu1_v)
    pltpu.sync_copy(cu2_hbm, cu2_v)

    iota = lax.iota(jnp.int32, 16)
    lo = jnp.minimum(iota, _B)
    hi = jnp.minimum(iota + 1, _B)
    s1v = plsc.load_gather(cu1_v, [lo])
    s2v = plsc.load_gather(cu2_v, [lo])
    l1v = plsc.load_gather(cu1_v, [hi]) - s1v
    l2v = plsc.load_gather(cu2_v, [hi]) - s2v

    over = (l1v + l2v) > _BUDGET
    t1 = jnp.where(l2v <= _HALF, _BUDGET - l2v,
                   jnp.where(l1v <= _HALF, l1v, _HALF))
    t2 = jnp.where(l2v <= _HALF, l2v,
                   jnp.where(l1v <= _HALF, _BUDGET - l1v, _BUDGET - _HALF))
    l1v = jnp.where(over, t1, l1v)
    l2v = jnp.where(over, t2, l2v)

    sel = iota == b
    l1 = jnp.sum(jnp.where(sel, l1v, 0))
    l2 = jnp.sum(jnp.where(sel, l2v, 0))
    s1 = jnp.sum(jnp.where(sel, s1v, 0))
    s2 = jnp.sum(jnp.where(sel, s2v, 0))

    w1 = pl.multiple_of(jnp.clip((s1 + base - 1) & -16, 0, _TOT - _TWIN), 16)
    w2 = pl.multiple_of(
        jnp.clip((s2 + base - l1 - 2) & -16, 0, _TOT - _TWIN), 16)
    c1 = pltpu.async_copy(tok1_hbm.at[pl.ds(w1, _TWIN)], tokw1, sem)
    c2 = pltpu.async_copy(tok2_hbm.at[pl.ds(w2, _TWIN)], tokw2, sem)

    for j in range(_CHUNK // 16):
        p = base + j * 16 + iota
        in1 = (p >= 1) & (p <= l1)
        in2 = (p >= l1 + 2) & (p <= l1 + 1 + l2)
        cidx_v[pl.ds(j * 16, 16)] = jnp.where(
            in1, s1 + p - 1,
            jnp.where(in2, _TOT + s2 + p - l1 - 2,
                      _ZBASE + j * 16 + iota))

    g = pltpu.async_copy(ctab_hbm.at[cidx_v], buf, sem)

    c1.wait()
    c2.wait()

    for j in range(_CHUNK // 16):
        p = base + j * 16 + iota
        in1 = (p >= 1) & (p <= l1)
        in2 = (p >= l1 + 2) & (p <= l1 + 1 + l2)
        sep = (p == l1 + 1) | (p == l1 + l2 + 2)
        lt1 = jnp.clip(s1 + p - 1 - w1, 0, _TWIN - 1)
        lt2 = jnp.clip(s2 + p - l1 - 2 - w2, 0, _TWIN - 1)
        t1g = plsc.load_gather(tokw1, [lt1])
        t2g = plsc.load_gather(tokw2, [lt2])
        ids = jnp.where(p == 0, _CLS,
                        jnp.where(sep, _SEP,
                                  jnp.where(in1, t1g,
                                            jnp.where(in2, t2g, _PAD))))
        ids_v[pl.ds(j * 16, 16)] = ids
        mask_v[pl.ds(j * 16, 16)] = (p < l1 + l2 + 3).astype(jnp.int32)
        types_v[pl.ds(j * 16, 16)] = (
            (p >= l1 + 2) & (p <= l1 + l2 + 2)).astype(jnp.int32)

    pltpu.sync_copy(ids_v, ids_hbm.at[b, pl.ds(base, _CHUNK)])
    pltpu.sync_copy(mask_v, mask_hbm.at[b, pl.ds(base, _CHUNK)])
    pltpu.sync_copy(types_v, types_hbm.at[b, pl.ds(base, _CHUNK)])

    g.wait()
    pltpu.sync_copy(buf, packed_hbm.at[b, pl.ds(base, _CHUNK)])


_sc_call = functools.partial(
    pl.kernel,
    out_type=(
        jax.ShapeDtypeStruct((_B, _SEQ), jnp.int32),
        jax.ShapeDtypeStruct((_B, _SEQ), jnp.int32),
        jax.ShapeDtypeStruct((_B, _SEQ), jnp.int32),
        jax.ShapeDtypeStruct((_B, _SEQ, _D), jnp.float32),
    ),
    mesh=plsc.VectorSubcoreMesh(
        core_axis_name="c", subcore_axis_name="s",
        num_cores=_NC, num_subcores=_NS),
    compiler_params=pltpu.CompilerParams(needs_layout_passes=False),
    scratch_types=[
        pltpu.VMEM((_B + 1,), jnp.int32),
        pltpu.VMEM((_B + 1,), jnp.int32),
        pltpu.VMEM((_TWIN,), jnp.int32),
        pltpu.VMEM((_TWIN,), jnp.int32),
        pltpu.VMEM((_CHUNK,), jnp.int32),
        pltpu.VMEM((_CHUNK, _D), jnp.float32),
        pltpu.VMEM((_CHUNK,), jnp.int32),
        pltpu.VMEM((_CHUNK,), jnp.int32),
        pltpu.VMEM((_CHUNK,), jnp.int32),
        pltpu.SemaphoreType.DMA,
    ],
)(_sc_body)


def kernel(tokens1, tokens2, feats1, feats2, cu_seqlens1, cu_seqlens2):
    ctab = jnp.concatenate(
        [feats1, feats2, jnp.zeros((_CHUNK, _D), jnp.float32)], axis=0)
    return _sc_call(tokens1.astype(jnp.int32), tokens2.astype(jnp.int32),
                    ctab,
                    cu_seqlens1.astype(jnp.int32),
                    cu_seqlens2.astype(jnp.int32))


# speedup vs baseline: 1.1201x; 1.0065x over previous
import functools

import jax
import jax.numpy as jnp
from jax import lax
from jax.experimental import pallas as pl
from jax.experimental.pallas import tpu as pltpu
from jax.experimental.pallas import tpu_sc as plsc

_B = 8
_SEQ = 512
_TOT = 2048
_D = 128
_CLS = 101
_SEP = 102
_PAD = 0

_NC = 2
_NS = 16
_NW = _NC * _NS
_CHUNK = _B * _SEQ // _NW
_CPB = _SEQ // _CHUNK
_TWIN = 160
_ZBASE = 2 * _TOT

_BUDGET = _SEQ - 3
_HALF = _BUDGET // 2


def _sc_body(tok1_hbm, tok2_hbm, ctab_hbm, cu1_hbm, cu2_hbm,
             ids_hbm, mask_hbm, types_hbm, packed_hbm,
             cu1_v, cu2_v, tokw1, tokw2, cidx_a, cidx_b,
             buf_a, buf_b, ids_v, mask_v, types_v,
             sem, sem_a, sem_b, wsem):
    wid = lax.axis_index("s") * _NC + lax.axis_index("c")
    b = wid // _CPB
    base = (wid % _CPB) * _CHUNK

    pltpu.sync_copy(cu1_hbm, cu1_v)
    pltpu.sync_copy(cu2_hbm, cu2_v)

    iota = lax.iota(jnp.int32, 16)
    lo = jnp.minimum(iota, _B)
    hi = jnp.minimum(iota + 1, _B)
    s1v = plsc.load_gather(cu1_v, [lo])
    s2v = plsc.load_gather(cu2_v, [lo])
    l1v = plsc.load_gather(cu1_v, [hi]) - s1v
    l2v = plsc.load_gather(cu2_v, [hi]) - s2v

    over = (l1v + l2v) > _BUDGET
    t1 = jnp.where(l2v <= _HALF, _BUDGET - l2v,
                   jnp.where(l1v <= _HALF, l1v, _HALF))
    t2 = jnp.where(l2v <= _HALF, l2v,
                   jnp.where(l1v <= _HALF, _BUDGET - l1v, _BUDGET - _HALF))
    l1v = jnp.where(over, t1, l1v)
    l2v = jnp.where(over, t2, l2v)

    sel = iota == b
    l1 = jnp.sum(jnp.where(sel, l1v, 0))
    l2 = jnp.sum(jnp.where(sel, l2v, 0))
    s1 = jnp.sum(jnp.where(sel, s1v, 0))
    s2 = jnp.sum(jnp.where(sel, s2v, 0))

    w1 = pl.multiple_of(jnp.clip((s1 + base - 1) & -16, 0, _TOT - _TWIN), 16)
    w2 = pl.multiple_of(
        jnp.clip((s2 + base - l1 - 2) & -16, 0, _TOT - _TWIN), 16)
    c1 = pltpu.async_copy(tok1_hbm.at[pl.ds(w1, _TWIN)], tokw1, sem)
    c2 = pltpu.async_copy(tok2_hbm.at[pl.ds(w2, _TWIN)], tokw2, sem)

    half = _CHUNK // 2
    for j in range(_CHUNK // 16):
        p = base + j * 16 + iota
        in1 = (p >= 1) & (p <= l1)
        in2 = (p >= l1 + 2) & (p <= l1 + 1 + l2)
        idx = jnp.where(
            in1, s1 + p - 1,
            jnp.where(in2, _TOT + s2 + p - l1 - 2,
                      _ZBASE + j * 16 + iota))
        if j < half // 16:
            cidx_a[pl.ds(j * 16, 16)] = idx
        else:
            cidx_b[pl.ds(j * 16 - half, 16)] = idx
        if j == half // 16 - 1:
            ga = pltpu.async_copy(ctab_hbm.at[cidx_a], buf_a, sem_a)

    gb = pltpu.async_copy(ctab_hbm.at[cidx_b], buf_b, sem_b)

    ga.wait()
    wa = pltpu.async_copy(buf_a, packed_hbm.at[b, pl.ds(base, half)], wsem)

    c1.wait()
    c2.wait()

    for j in range(_CHUNK // 16):
        p = base + j * 16 + iota
        in1 = (p >= 1) & (p <= l1)
        in2 = (p >= l1 + 2) & (p <= l1 + 1 + l2)
        sep = (p == l1 + 1) | (p == l1 + l2 + 2)
        lt1 = jnp.clip(s1 + p - 1 - w1, 0, _TWIN - 1)
        lt2 = jnp.clip(s2 + p - l1 - 2 - w2, 0, _TWIN - 1)
        t1g = plsc.load_gather(tokw1, [lt1])
        t2g = plsc.load_gather(tokw2, [lt2])
        ids = jnp.where(p == 0, _CLS,
                        jnp.where(sep, _SEP,
                                  jnp.where(in1, t1g,
                                            jnp.where(in2, t2g, _PAD))))
        ids_v[pl.ds(j * 16, 16)] = ids
        mask_v[pl.ds(j * 16, 16)] = (p < l1 + l2 + 3).astype(jnp.int32)
        types_v[pl.ds(j * 16, 16)] = (
            (p >= l1 + 2) & (p <= l1 + l2 + 2)).astype(jnp.int32)

    gb.wait()
    wb = pltpu.async_copy(
        buf_b, packed_hbm.at[b, pl.ds(base + half, half)], wsem)

    pltpu.sync_copy(ids_v, ids_hbm.at[b, pl.ds(base, _CHUNK)])
    pltpu.sync_copy(mask_v, mask_hbm.at[b, pl.ds(base, _CHUNK)])
    pltpu.sync_copy(types_v, types_hbm.at[b, pl.ds(base, _CHUNK)])

    wa.wait()
    wb.wait()


_sc_call = functools.partial(
    pl.kernel,
    out_type=(
        jax.ShapeDtypeStruct((_B, _SEQ), jnp.int32),
        jax.ShapeDtypeStruct((_B, _SEQ), jnp.int32),
        jax.ShapeDtypeStruct((_B, _SEQ), jnp.int32),
        jax.ShapeDtypeStruct((_B, _SEQ, _D), jnp.float32),
    ),
    mesh=plsc.VectorSubcoreMesh(
        core_axis_name="c", subcore_axis_name="s",
        num_cores=_NC, num_subcores=_NS),
    compiler_params=pltpu.CompilerParams(needs_layout_passes=False),
    scratch_types=[
        pltpu.VMEM((_B + 1,), jnp.int32),
        pltpu.VMEM((_B + 1,), jnp.int32),
        pltpu.VMEM((_TWIN,), jnp.int32),
        pltpu.VMEM((_TWIN,), jnp.int32),
        pltpu.VMEM((_CHUNK // 2,), jnp.int32),
        pltpu.VMEM((_CHUNK // 2,), jnp.int32),
        pltpu.VMEM((_CHUNK // 2, _D), jnp.float32),
        pltpu.VMEM((_CHUNK // 2, _D), jnp.float32),
        pltpu.VMEM((_CHUNK,), jnp.int32),
        pltpu.VMEM((_CHUNK,), jnp.int32),
        pltpu.VMEM((_CHUNK,), jnp.int32),
        pltpu.SemaphoreType.DMA,
        pltpu.SemaphoreType.DMA,
        pltpu.SemaphoreType.DMA,
        pltpu.SemaphoreType.DMA,
    ],
)(_sc_body)


def kernel(tokens1, tokens2, feats1, feats2, cu_seqlens1, cu_seqlens2):
    ctab = jnp.concatenate(
        [feats1, feats2, jnp.zeros((_CHUNK, _D), jnp.float32)], axis=0)
    return _sc_call(tokens1.astype(jnp.int32), tokens2.astype(jnp.int32),
                    ctab,
                    cu_seqlens1.astype(jnp.int32),
                    cu_seqlens2.astype(jnp.int32))


# speedup vs baseline: 1.1273x; 1.0064x over previous
import functools

import jax
import jax.numpy as jnp
from jax import lax
from jax.experimental import pallas as pl
from jax.experimental.pallas import tpu as pltpu
from jax.experimental.pallas import tpu_sc as plsc

_B = 8
_SEQ = 512
_TOT = 2048
_D = 128
_CLS = 101
_SEP = 102
_PAD = 0

_NC = 2
_NS = 16
_NW = _NC * _NS
_CHUNK = _B * _SEQ // _NW
_CPB = _SEQ // _CHUNK
_TWIN = 160
_ZBASE = 2 * _TOT

_BUDGET = _SEQ - 3
_HALF = _BUDGET // 2


def _sc_body(tok1_hbm, tok2_hbm, ctab_hbm, cu1_hbm, cu2_hbm,
             ids_hbm, mask_hbm, types_hbm, packed_hbm,
             cu1_v, cu2_v, tokw1, tokw2, cidx_a, cidx_b,
             buf_a, buf_b, ids_v, mask_v, types_v,
             sem, sem_a, sem_b, wsem):
    wid = lax.axis_index("s") * _NC + lax.axis_index("c")
    b = wid // _CPB
    base = (wid % _CPB) * _CHUNK

    pltpu.sync_copy(cu1_hbm, cu1_v)
    pltpu.sync_copy(cu2_hbm, cu2_v)

    iota = lax.iota(jnp.int32, 16)
    lo = jnp.minimum(iota, _B)
    hi = jnp.minimum(iota + 1, _B)
    s1v = plsc.load_gather(cu1_v, [lo])
    s2v = plsc.load_gather(cu2_v, [lo])
    l1v = plsc.load_gather(cu1_v, [hi]) - s1v
    l2v = plsc.load_gather(cu2_v, [hi]) - s2v

    over = (l1v + l2v) > _BUDGET
    t1 = jnp.where(l2v <= _HALF, _BUDGET - l2v,
                   jnp.where(l1v <= _HALF, l1v, _HALF))
    t2 = jnp.where(l2v <= _HALF, l2v,
                   jnp.where(l1v <= _HALF, _BUDGET - l1v, _BUDGET - _HALF))
    l1v = jnp.where(over, t1, l1v)
    l2v = jnp.where(over, t2, l2v)

    sel = iota == b
    l1 = jnp.sum(jnp.where(sel, l1v, 0))
    l2 = jnp.sum(jnp.where(sel, l2v, 0))
    s1 = jnp.sum(jnp.where(sel, s1v, 0))
    s2 = jnp.sum(jnp.where(sel, s2v, 0))

    w1 = pl.multiple_of(jnp.clip((s1 + base - 1) & -16, 0, _TOT - _TWIN), 16)
    w2 = pl.multiple_of(
        jnp.clip((s2 + base - l1 - 2) & -16, 0, _TOT - _TWIN), 16)
    c1 = pltpu.async_copy(tok1_hbm.at[pl.ds(w1, _TWIN)], tokw1, sem)
    c2 = pltpu.async_copy(tok2_hbm.at[pl.ds(w2, _TWIN)], tokw2, sem)

    half = _CHUNK // 2
    for j in range(_CHUNK // 16):
        p = base + j * 16 + iota
        in1 = (p >= 1) & (p <= l1)
        in2 = (p >= l1 + 2) & (p <= l1 + 1 + l2)
        idx = jnp.where(
            in1, s1 + p - 1,
            jnp.where(in2, _TOT + s2 + p - l1 - 2,
                      _ZBASE + j * 16 + iota))
        if j < half // 16:
            cidx_a[pl.ds(j * 16, 16)] = idx
        else:
            cidx_b[pl.ds(j * 16 - half, 16)] = idx
        if j == half // 16 - 1:
            ga = pltpu.async_copy(ctab_hbm.at[cidx_a], buf_a, sem_a)

    gb = pltpu.async_copy(ctab_hbm.at[cidx_b], buf_b, sem_b)

    ga.wait()
    wa = pltpu.async_copy(buf_a, packed_hbm.at[b, pl.ds(base, half)], wsem)

    c1.wait()
    c2.wait()

    for j in range(_CHUNK // 16):
        p = base + j * 16 + iota
        in1 = (p >= 1) & (p <= l1)
        in2 = (p >= l1 + 2) & (p <= l1 + 1 + l2)
        sep = (p == l1 + 1) | (p == l1 + l2 + 2)
        lt1 = jnp.clip(s1 + p - 1 - w1, 0, _TWIN - 1)
        lt2 = jnp.clip(s2 + p - l1 - 2 - w2, 0, _TWIN - 1)
        t1g = plsc.load_gather(tokw1, [lt1])
        t2g = plsc.load_gather(tokw2, [lt2])
        ids = jnp.where(p == 0, _CLS,
                        jnp.where(sep, _SEP,
                                  jnp.where(in1, t1g,
                                            jnp.where(in2, t2g, _PAD))))
        ids_v[pl.ds(j * 16, 16)] = ids
        mask_v[pl.ds(j * 16, 16)] = (p < l1 + l2 + 3).astype(jnp.int32)
        types_v[pl.ds(j * 16, 16)] = (
            (p >= l1 + 2) & (p <= l1 + l2 + 2)).astype(jnp.int32)

    gb.wait()
    wb = pltpu.async_copy(
        buf_b, packed_hbm.at[b, pl.ds(base + half, half)], wsem)

    ci = pltpu.async_copy(ids_v, ids_hbm.at[b, pl.ds(base, _CHUNK)], sem)
    cm = pltpu.async_copy(mask_v, mask_hbm.at[b, pl.ds(base, _CHUNK)], sem)
    ct = pltpu.async_copy(types_v, types_hbm.at[b, pl.ds(base, _CHUNK)], sem)

    wa.wait()
    wb.wait()
    ci.wait()
    cm.wait()
    ct.wait()


_sc_call = functools.partial(
    pl.kernel,
    out_type=(
        jax.ShapeDtypeStruct((_B, _SEQ), jnp.int32),
        jax.ShapeDtypeStruct((_B, _SEQ), jnp.int32),
        jax.ShapeDtypeStruct((_B, _SEQ), jnp.int32),
        jax.ShapeDtypeStruct((_B, _SEQ, _D), jnp.float32),
    ),
    mesh=plsc.VectorSubcoreMesh(
        core_axis_name="c", subcore_axis_name="s",
        num_cores=_NC, num_subcores=_NS),
    compiler_params=pltpu.CompilerParams(needs_layout_passes=False),
    scratch_types=[
        pltpu.VMEM((_B + 1,), jnp.int32),
        pltpu.VMEM((_B + 1,), jnp.int32),
        pltpu.VMEM((_TWIN,), jnp.int32),
        pltpu.VMEM((_TWIN,), jnp.int32),
        pltpu.VMEM((_CHUNK // 2,), jnp.int32),
        pltpu.VMEM((_CHUNK // 2,), jnp.int32),
        pltpu.VMEM((_CHUNK // 2, _D), jnp.float32),
        pltpu.VMEM((_CHUNK // 2, _D), jnp.float32),
        pltpu.VMEM((_CHUNK,), jnp.int32),
        pltpu.VMEM((_CHUNK,), jnp.int32),
        pltpu.VMEM((_CHUNK,), jnp.int32),
        pltpu.SemaphoreType.DMA,
        pltpu.SemaphoreType.DMA,
        pltpu.SemaphoreType.DMA,
        pltpu.SemaphoreType.DMA,
    ],
)(_sc_body)


def kernel(tokens1, tokens2, feats1, feats2, cu_seqlens1, cu_seqlens2):
    ctab = jnp.concatenate(
        [feats1, feats2, jnp.zeros((_CHUNK, _D), jnp.float32)], axis=0)
    return _sc_call(tokens1.astype(jnp.int32), tokens2.astype(jnp.int32),
                    ctab,
                    cu_seqlens1.astype(jnp.int32),
                    cu_seqlens2.astype(jnp.int32))


# speedup vs baseline: 1.1475x; 1.0179x over previous
import functools

import jax
import jax.numpy as jnp
from jax import lax
from jax.experimental import pallas as pl
from jax.experimental.pallas import tpu as pltpu
from jax.experimental.pallas import tpu_sc as plsc

_B = 8
_SEQ = 512
_TOT = 2048
_D = 128
_CLS = 101
_SEP = 102
_PAD = 0

_NC = 2
_NS = 16
_NW = _NC * _NS
_CHUNK = _B * _SEQ // _NW
_CPB = _SEQ // _CHUNK
_TWIN = 160
_ZBASE = 2 * _TOT

_BUDGET = _SEQ - 3
_HALF = _BUDGET // 2


def _sc_body(tok1_hbm, tok2_hbm, ctab_hbm, cu1_hbm, cu2_hbm,
             ids_hbm, mask_hbm, types_hbm, packed_hbm,
             cu1_v, cu2_v, tokw1, tokw2, cidx_a, cidx_b,
             buf_a, buf_b, ids_v, mask_v, types_v,
             sem, sem_a, sem_b, wsem):
    wid = lax.axis_index("s") * _NC + lax.axis_index("c")
    b = wid // _CPB
    base = (wid % _CPB) * _CHUNK

    k1 = pltpu.async_copy(cu1_hbm, cu1_v, sem_a)
    k2 = pltpu.async_copy(cu2_hbm, cu2_v, sem_b)
    k1.wait()
    k2.wait()

    iota = lax.iota(jnp.int32, 16)
    lo = jnp.minimum(iota, _B)
    hi = jnp.minimum(iota + 1, _B)
    s1v = plsc.load_gather(cu1_v, [lo])
    s2v = plsc.load_gather(cu2_v, [lo])
    l1v = plsc.load_gather(cu1_v, [hi]) - s1v
    l2v = plsc.load_gather(cu2_v, [hi]) - s2v

    over = (l1v + l2v) > _BUDGET
    t1 = jnp.where(l2v <= _HALF, _BUDGET - l2v,
                   jnp.where(l1v <= _HALF, l1v, _HALF))
    t2 = jnp.where(l2v <= _HALF, l2v,
                   jnp.where(l1v <= _HALF, _BUDGET - l1v, _BUDGET - _HALF))
    l1v = jnp.where(over, t1, l1v)
    l2v = jnp.where(over, t2, l2v)

    sel = iota == b
    l1 = jnp.sum(jnp.where(sel, l1v, 0))
    l2 = jnp.sum(jnp.where(sel, l2v, 0))
    s1 = jnp.sum(jnp.where(sel, s1v, 0))
    s2 = jnp.sum(jnp.where(sel, s2v, 0))

    w1 = pl.multiple_of(jnp.clip((s1 + base - 1) & -16, 0, _TOT - _TWIN), 16)
    w2 = pl.multiple_of(
        jnp.clip((s2 + base - l1 - 2) & -16, 0, _TOT - _TWIN), 16)
    c1 = pltpu.async_copy(tok1_hbm.at[pl.ds(w1, _TWIN)], tokw1, sem)
    c2 = pltpu.async_copy(tok2_hbm.at[pl.ds(w2, _TWIN)], tokw2, sem)

    half = _CHUNK // 2
    for j in range(_CHUNK // 16):
        p = base + j * 16 + iota
        in1 = (p >= 1) & (p <= l1)
        in2 = (p >= l1 + 2) & (p <= l1 + 1 + l2)
        idx = jnp.where(
            in1, s1 + p - 1,
            jnp.where(in2, _TOT + s2 + p - l1 - 2,
                      _ZBASE + j * 16 + iota))
        if j < half // 16:
            cidx_a[pl.ds(j * 16, 16)] = idx
        else:
            cidx_b[pl.ds(j * 16 - half, 16)] = idx
        if j == half // 16 - 1:
            ga = pltpu.async_copy(ctab_hbm.at[cidx_a], buf_a, sem_a)

    gb = pltpu.async_copy(ctab_hbm.at[cidx_b], buf_b, sem_b)

    ga.wait()
    wa = pltpu.async_copy(buf_a, packed_hbm.at[b, pl.ds(base, half)], wsem)

    c1.wait()
    c2.wait()

    for j in range(_CHUNK // 16):
        p = base + j * 16 + iota
        in1 = (p >= 1) & (p <= l1)
        in2 = (p >= l1 + 2) & (p <= l1 + 1 + l2)
        sep = (p == l1 + 1) | (p == l1 + l2 + 2)
        lt1 = jnp.clip(s1 + p - 1 - w1, 0, _TWIN - 1)
        lt2 = jnp.clip(s2 + p - l1 - 2 - w2, 0, _TWIN - 1)
        t1g = plsc.load_gather(tokw1, [lt1])
        t2g = plsc.load_gather(tokw2, [lt2])
        ids = jnp.where(p == 0, _CLS,
                        jnp.where(sep, _SEP,
                                  jnp.where(in1, t1g,
                                            jnp.where(in2, t2g, _PAD))))
        ids_v[pl.ds(j * 16, 16)] = ids
        mask_v[pl.ds(j * 16, 16)] = (p < l1 + l2 + 3).astype(jnp.int32)
        types_v[pl.ds(j * 16, 16)] = (
            (p >= l1 + 2) & (p <= l1 + l2 + 2)).astype(jnp.int32)

    gb.wait()
    wb = pltpu.async_copy(
        buf_b, packed_hbm.at[b, pl.ds(base + half, half)], wsem)

    ci = pltpu.async_copy(ids_v, ids_hbm.at[b, pl.ds(base, _CHUNK)], sem)
    cm = pltpu.async_copy(mask_v, mask_hbm.at[b, pl.ds(base, _CHUNK)], sem)
    ct = pltpu.async_copy(types_v, types_hbm.at[b, pl.ds(base, _CHUNK)], sem)

    wa.wait()
    wb.wait()
    ci.wait()
    cm.wait()
    ct.wait()


_sc_call = functools.partial(
    pl.kernel,
    out_type=(
        jax.ShapeDtypeStruct((_B, _SEQ), jnp.int32),
        jax.ShapeDtypeStruct((_B, _SEQ), jnp.int32),
        jax.ShapeDtypeStruct((_B, _SEQ), jnp.int32),
        jax.ShapeDtypeStruct((_B, _SEQ, _D), jnp.float32),
    ),
    mesh=plsc.VectorSubcoreMesh(
        core_axis_name="c", subcore_axis_name="s",
        num_cores=_NC, num_subcores=_NS),
    compiler_params=pltpu.CompilerParams(needs_layout_passes=False),
    scratch_types=[
        pltpu.VMEM((_B + 1,), jnp.int32),
        pltpu.VMEM((_B + 1,), jnp.int32),
        pltpu.VMEM((_TWIN,), jnp.int32),
        pltpu.VMEM((_TWIN,), jnp.int32),
        pltpu.VMEM((_CHUNK // 2,), jnp.int32),
        pltpu.VMEM((_CHUNK // 2,), jnp.int32),
        pltpu.VMEM((_CHUNK // 2, _D), jnp.float32),
        pltpu.VMEM((_CHUNK // 2, _D), jnp.float32),
        pltpu.VMEM((_CHUNK,), jnp.int32),
        pltpu.VMEM((_CHUNK,), jnp.int32),
        pltpu.VMEM((_CHUNK,), jnp.int32),
        pltpu.SemaphoreType.DMA,
        pltpu.SemaphoreType.DMA,
        pltpu.SemaphoreType.DMA,
        pltpu.SemaphoreType.DMA,
    ],
)(_sc_body)


def kernel(tokens1, tokens2, feats1, feats2, cu_seqlens1, cu_seqlens2):
    ctab = jnp.concatenate(
        [feats1, feats2, jnp.zeros((_CHUNK, _D), jnp.float32)], axis=0)
    return _sc_call(tokens1.astype(jnp.int32), tokens2.astype(jnp.int32),
                    ctab,
                    cu_seqlens1.astype(jnp.int32),
                    cu_seqlens2.astype(jnp.int32))
